# baseline (speedup 1.0000x reference)
import jax
import jax.numpy as jnp
from jax.experimental import pallas as pl
from jax.experimental.pallas import tpu as pltpu


def _copy_kernel(x_ref, o_ref):
    o_ref[...] = x_ref[...]


def kernel(method_soft, label, features):
    del label, features
    n, d = method_soft.shape
    total = n * d
    if total % 128 == 0:
        shape = (total // 128, 128)
    else:
        shape = (n, d)
    x = method_soft.reshape(shape)
    y = pl.pallas_call(
        _copy_kernel,
        out_shape=jax.ShapeDtypeStruct(shape, method_soft.dtype),
    )(x)
    return y.reshape(n, d)
